# Initial kernel scaffold; baseline (speedup 1.0000x reference)
import jax
import jax.numpy as jnp
from jax.experimental import pallas as pl


def kernel(W_user, W_item, spk_emb, edge_values, user_fea_index, item_fea_index, item_fea_mask, edge_index):
    raise NotImplementedError("write your pallas kernel here")


# speedup vs baseline: 2.0824x; 2.0824x over previous
import functools

import jax
import jax.numpy as jnp
from jax import lax
from jax.experimental import pallas as pl
from jax.experimental.pallas import tpu as pltpu
from jax.experimental.pallas import tpu_sc as plsc

_N_USERS = 50000
_N_ITEMS = 10000
_D = 64
_N_NODES = _N_USERS + _N_ITEMS
_N_EDGES = 960000

_NC = 2
_NS = 16
_NW = _NC * _NS

_HALF = _N_NODES // _NC
_ACC_ROWS = 30016
_ZCH = _ACC_ROWS // _NS

_ECOL = 96
_EROWS = _N_EDGES // _ECOL

_UCH = 80
_NUCH = _N_USERS // _UCH
_ICH = 80
_NICH = _N_ITEMS // _ICH

_mesh = plsc.VectorSubcoreMesh(core_axis_name="c", subcore_axis_name="s")


def _wid():
    return lax.axis_index("c") * _NS + lax.axis_index("s")


@functools.partial(
    pl.kernel,
    mesh=_mesh,
    out_type=jax.ShapeDtypeStruct((_N_NODES, _D), jnp.float32),
    compiler_params=pltpu.CompilerParams(use_tc_tiling_on_sc=False),
    scratch_types=[
        pltpu.VMEM((6, _UCH), jnp.int32),
        pltpu.VMEM((6, _ICH), jnp.int32),
        pltpu.VMEM((6 * _UCH, _D), jnp.float32),
        pltpu.VMEM((_UCH, _D), jnp.float32),
        pltpu.SemaphoreType.DMA,
    ],
)
def _emb0_kernel(wu_hbm, ufi_hbm, spk_hbm, wi_hbm, ifi_hbm, imask_hbm,
                 out_hbm, idx_v, msk_v, g_v, o_v, sem):
    w = _wid()

    n_uch = 19 + (w < 17).astype(jnp.int32)

    def user_chunk(k, _):
        c = w + _NW * k
        base = c * _UCH
        for f in range(5):
            pltpu.sync_copy(ufi_hbm.at[f * _NUCH + c], idx_v.at[f])
        cps = [pltpu.async_copy(wu_hbm.at[idx_v.at[f]],
                                g_v.at[pl.ds(f * _UCH, _UCH)], sem)
               for f in range(5)]
        pltpu.sync_copy(spk_hbm.at[pl.ds(base, _UCH)], o_v)
        for cp in cps:
            cp.wait()

        def row_body(r, _):
            for q in range(4):
                s = pl.ds(16 * q, 16)
                acc = g_v[r, s]
                for f in range(1, 5):
                    acc = acc + g_v[f * _UCH + r, s]
                o_v[r, s] = acc * 0.1 + o_v[r, s] * 0.5
            return 0

        lax.fori_loop(0, _UCH, row_body, 0)
        pltpu.sync_copy(o_v, out_hbm.at[pl.ds(base, _UCH)])
        return 0

    lax.fori_loop(0, n_uch, user_chunk, 0)

    n_ich = 3 + (w < 29).astype(jnp.int32)

    def item_chunk(k, _):
        c = w + _NW * k
        base = c * _ICH
        for f in range(6):
            pltpu.sync_copy(ifi_hbm.at[f * _NICH + c], idx_v.at[f])
            pltpu.sync_copy(imask_hbm.at[f * _NICH + c], msk_v.at[f])
        for f in range(6):
            for q in range(5):
                s = pl.ds(16 * q, 16)
                idx_v[f, s] = idx_v[f, s] * msk_v[f, s]
        cps = [pltpu.async_copy(wi_hbm.at[idx_v.at[f]],
                                g_v.at[pl.ds(f * _ICH, _ICH)], sem)
               for f in range(6)]
        for cp in cps:
            cp.wait()

        def row_body(r, _):
            for q in range(4):
                s = pl.ds(16 * q, 16)
                acc = g_v[r, s]
                for f in range(1, 6):
                    acc = acc + g_v[f * _ICH + r, s]
                o_v[r, s] = acc * (1.0 / 6.0)
            return 0

        lax.fori_loop(0, _ICH, row_body, 0)
        pltpu.sync_copy(o_v, out_hbm.at[pl.ds(_N_USERS + base, _ICH)])
        return 0

    lax.fori_loop(0, n_ich, item_chunk, 0)


_SUP = 5


@functools.partial(
    pl.kernel,
    mesh=_mesh,
    out_type=jax.ShapeDtypeStruct((_N_NODES, _D), jnp.float32),
    compiler_params=pltpu.CompilerParams(use_tc_tiling_on_sc=False),
    scratch_types=[
        pltpu.VMEM((_SUP, _ECOL), jnp.int32),
        pltpu.VMEM((_SUP, _ECOL), jnp.int32),
        pltpu.VMEM((_SUP, _ECOL), jnp.float32),
        pltpu.VMEM((_ECOL, _D), jnp.float32),
        pltpu.VMEM_SHARED((_ACC_ROWS, _D), jnp.float32),
        pltpu.SemaphoreType.DMA,
    ],
)
def _layer_kernel(emb_hbm, src_hbm, dst_hbm, val_hbm, zeros_hbm, out_hbm,
                  sidx_v, ldst_v, val_v, rows_v, acc_s, sem):
    cid = lax.axis_index("c")
    sid = lax.axis_index("s")
    half_base = cid * _HALF

    pltpu.sync_copy(zeros_hbm, acc_s.at[pl.ds(sid * _ZCH, _ZCH)])
    plsc.subcore_barrier()

    row0 = sid * 625

    def slab_body(si, _):
        r0 = row0 + si * _SUP
        pltpu.sync_copy(src_hbm.at[pl.ds(r0, _SUP)], sidx_v)
        pltpu.sync_copy(dst_hbm.at[pl.ds(r0, _SUP)], ldst_v)
        pltpu.sync_copy(val_hbm.at[pl.ds(r0, _SUP)], val_v)

        def ld_body(j, _):
            for t in range(6):
                s = pl.ds(16 * t, 16)
                lv = ldst_v[j, s] - half_base
                ok = (lv >= 0) & (lv < _HALF)
                ldst_v[j, s] = jnp.where(ok, lv, _HALF)
            return 0

        lax.fori_loop(0, _SUP, ld_body, 0)

        def chunk_body(j, _):
            pltpu.async_copy(emb_hbm.at[sidx_v.at[j]], rows_v, sem).wait()

            def grp16(t, _):
                vv = val_v[j, pl.ds(16 * t, 16)]
                for u in range(16):
                    e = t * 16 + u
                    v = vv[u]
                    for q in range(4):
                        s = pl.ds(16 * q, 16)
                        rows_v[e, s] = rows_v[e, s] * v
                return 0

            lax.fori_loop(0, _ECOL // 16, grp16, 0)
            pltpu.sync_copy(rows_v, acc_s.at[ldst_v.at[j]], add=True)
            return 0

        lax.fori_loop(0, _SUP, chunk_body, 0)
        return 0

    lax.fori_loop(0, 625 // _SUP, slab_body, 0)
    plsc.subcore_barrier()

    r = sid * 1875
    pltpu.sync_copy(acc_s.at[pl.ds(r, 1875)],
                    out_hbm.at[pl.ds(half_base + r, 1875)])


def _combine_body(a_ref, b_ref, c_ref, d_ref, o_ref):
    o_ref[...] = (a_ref[...] + b_ref[...] + c_ref[...] + d_ref[...]) * 0.25


_combine = pl.pallas_call(
    _combine_body,
    grid=(30,),
    in_specs=[pl.BlockSpec((2000, _D), lambda i: (i, 0))] * 4,
    out_specs=pl.BlockSpec((2000, _D), lambda i: (i, 0)),
    out_shape=jax.ShapeDtypeStruct((_N_NODES, _D), jnp.float32),
)


def kernel(W_user, W_item, spk_emb, edge_values, user_fea_index,
           item_fea_index, item_fea_mask, edge_index):
    f32 = jnp.float32
    i32 = jnp.int32
    ufi = user_fea_index.astype(i32).T.reshape(_NUCH * 5, _UCH)
    ifi = item_fea_index.astype(i32).T.reshape(_NICH * 6, _ICH)
    imask = item_fea_mask.astype(i32).T.reshape(_NICH * 6, _ICH)
    src = edge_index[0].astype(i32).reshape(_EROWS, _ECOL)
    dst = edge_index[1].astype(i32).reshape(_EROWS, _ECOL)
    val = edge_values.astype(f32).reshape(_EROWS, _ECOL)

    zeros = jnp.zeros((_ZCH, _D), f32)

    e0 = _emb0_kernel(W_user.astype(f32), ufi, spk_emb.astype(f32),
                      W_item.astype(f32), ifi, imask)
    e1 = _layer_kernel(e0, src, dst, val, zeros)
    e2 = _layer_kernel(e1, src, dst, val, zeros)
    e3 = _layer_kernel(e2, src, dst, val, zeros)
    out = _combine(e0, e1, e2, e3)
    return out[:_N_USERS], out[_N_USERS:]


# speedup vs baseline: 2.9799x; 1.4310x over previous
import functools

import jax
import jax.numpy as jnp
from jax import lax
from jax.experimental import pallas as pl
from jax.experimental.pallas import tpu as pltpu
from jax.experimental.pallas import tpu_sc as plsc

_N_USERS = 50000
_N_ITEMS = 10000
_D = 64
_DH = 32
_N_NODES = _N_USERS + _N_ITEMS
_N_EDGES = 960000

_NC = 2
_NS = 16
_NW = _NC * _NS

_HALF = _N_NODES // _NC
_ACC_ROWS = 30016
_ZCH = _ACC_ROWS // _NS

_ECOL = 96
_EROWS = _N_EDGES // _ECOL
_SUP = 5
_TROWS = _EROWS // _NS
_NSLABS = _TROWS // _SUP

_UCH = 80
_NUCH = _N_USERS // _UCH
_ICH = 80
_NICH = _N_ITEMS // _ICH

_mesh = plsc.VectorSubcoreMesh(core_axis_name="c", subcore_axis_name="s")


def _wid():
    return lax.axis_index("c") * _NS + lax.axis_index("s")


@functools.partial(
    pl.kernel,
    mesh=_mesh,
    out_type=jax.ShapeDtypeStruct((2 * _N_NODES, _DH), jnp.float32),
    compiler_params=pltpu.CompilerParams(use_tc_tiling_on_sc=False),
    scratch_types=[
        pltpu.VMEM((6, _UCH), jnp.int32),
        pltpu.VMEM((6, _ICH), jnp.int32),
        pltpu.VMEM((6 * _UCH, _D), jnp.float32),
        pltpu.VMEM((_UCH, _D), jnp.float32),
        pltpu.VMEM((_UCH, _DH), jnp.float32),
        pltpu.VMEM((_UCH, _DH), jnp.float32),
        pltpu.SemaphoreType.DMA,
    ],
)
def _emb0_kernel(wu_hbm, ufi_hbm, spk_hbm, wi_hbm, ifi_hbm, imask_hbm,
                 out_hbm, idx_v, msk_v, g_v, spk_v, o0_v, o1_v, sem):
    w = _wid()
    obufs = (o0_v, o1_v)

    n_uch = 19 + (w < 17).astype(jnp.int32)

    def user_chunk(k, _):
        c = w + _NW * k
        base = c * _UCH
        for f in range(5):
            pltpu.sync_copy(ufi_hbm.at[f * _NUCH + c], idx_v.at[f])
        cps = [pltpu.async_copy(wu_hbm.at[idx_v.at[f]],
                                g_v.at[pl.ds(f * _UCH, _UCH)], sem)
               for f in range(5)]
        pltpu.sync_copy(spk_hbm.at[pl.ds(base, _UCH)], spk_v)
        for cp in cps:
            cp.wait()

        def row_body(r, _):
            for q in range(4):
                s = pl.ds(16 * q, 16)
                acc = g_v[r, s]
                for f in range(1, 5):
                    acc = acc + g_v[f * _UCH + r, s]
                obufs[q // 2][r, pl.ds(16 * (q % 2), 16)] = (
                    acc * 0.1 + spk_v[r, s] * 0.5)
            return 0

        lax.fori_loop(0, _UCH, row_body, 0)
        pltpu.sync_copy(o0_v, out_hbm.at[pl.ds(base, _UCH)])
        pltpu.sync_copy(o1_v, out_hbm.at[pl.ds(_N_NODES + base, _UCH)])
        return 0

    lax.fori_loop(0, n_uch, user_chunk, 0)

    n_ich = 3 + (w < 29).astype(jnp.int32)

    def item_chunk(k, _):
        c = w + _NW * k
        base = c * _ICH
        for f in range(6):
            pltpu.sync_copy(ifi_hbm.at[f * _NICH + c], idx_v.at[f])
            pltpu.sync_copy(imask_hbm.at[f * _NICH + c], msk_v.at[f])
        for f in range(6):
            for q in range(5):
                s = pl.ds(16 * q, 16)
                idx_v[f, s] = idx_v[f, s] * msk_v[f, s]
        cps = [pltpu.async_copy(wi_hbm.at[idx_v.at[f]],
                                g_v.at[pl.ds(f * _ICH, _ICH)], sem)
               for f in range(6)]
        for cp in cps:
            cp.wait()

        def row_body(r, _):
            for q in range(4):
                s = pl.ds(16 * q, 16)
                acc = g_v[r, s]
                for f in range(1, 6):
                    acc = acc + g_v[f * _ICH + r, s]
                obufs[q // 2][r, pl.ds(16 * (q % 2), 16)] = acc * (1.0 / 6.0)
            return 0

        lax.fori_loop(0, _ICH, row_body, 0)
        pltpu.sync_copy(o0_v, out_hbm.at[pl.ds(_N_USERS + base, _ICH)])
        pltpu.sync_copy(o1_v,
                        out_hbm.at[pl.ds(_N_NODES + _N_USERS + base, _ICH)])
        return 0

    lax.fori_loop(0, n_ich, item_chunk, 0)


@functools.partial(
    pl.kernel,
    mesh=_mesh,
    out_type=jax.ShapeDtypeStruct((2 * _N_NODES, _DH), jnp.float32),
    compiler_params=pltpu.CompilerParams(use_tc_tiling_on_sc=False),
    scratch_types=[
        pltpu.VMEM((2, _SUP, _ECOL), jnp.int32),
        pltpu.VMEM((2, _SUP, _ECOL), jnp.int32),
        pltpu.VMEM((2, _SUP, _ECOL), jnp.float32),
        pltpu.VMEM((_SUP, _ECOL), jnp.int32),
        pltpu.VMEM((_ECOL, _DH), jnp.float32),
        pltpu.VMEM((_ECOL, _DH), jnp.float32),
        pltpu.VMEM((_ECOL, _DH), jnp.float32),
        pltpu.VMEM((_ECOL, _DH), jnp.float32),
        pltpu.VMEM((_ECOL, _DH), jnp.float32),
        pltpu.VMEM_SHARED((_ACC_ROWS, _DH), jnp.float32),
        pltpu.SemaphoreType.DMA,
        pltpu.SemaphoreType.DMA,
        pltpu.SemaphoreType.DMA,
    ],
)
def _layer_kernel(emb_hbm, src_hbm, dst_hbm, val_hbm, zeros_hbm, out_hbm,
                  sidx_v, dslab_v, val_v, ldst_v, r0_v, r1_v, r2_v, r3_v, r4_v,
                  acc_s, gsem, ssem, lsem):
    cid = lax.axis_index("c")
    sid = lax.axis_index("s")
    half_base = cid * _HALF
    bufs = [r0_v, r1_v, r2_v, r3_v, r4_v]
    row0 = sid * _TROWS

    def _slab_load(si, sb):
        r0 = row0 + si * _SUP
        pltpu.async_copy(src_hbm.at[pl.ds(r0, _SUP)], sidx_v.at[sb], lsem)
        pltpu.async_copy(dst_hbm.at[pl.ds(r0, _SUP)], dslab_v.at[sb], lsem)
        pltpu.async_copy(val_hbm.at[pl.ds(r0, _SUP)], val_v.at[sb], lsem)

    def _slab_wait(sb, hoff):
        pltpu.make_async_copy(src_hbm.at[pl.ds(0, _SUP)], sidx_v.at[0], lsem).wait()
        pltpu.make_async_copy(dst_hbm.at[pl.ds(0, _SUP)], dslab_v.at[0], lsem).wait()
        pltpu.make_async_copy(val_hbm.at[pl.ds(0, _SUP)], val_v.at[0], lsem).wait()
        for j in range(_SUP):
            for t in range(_ECOL // 16):
                s = pl.ds(16 * t, 16)
                sidx_v[sb, j, s] = sidx_v[sb, j, s] + hoff

    def _gwait():
        pltpu.make_async_copy(emb_hbm.at[pl.ds(0, _ECOL)], r0_v, gsem).wait()

    def _swait():
        pltpu.make_async_copy(emb_hbm.at[pl.ds(0, _ECOL)], r0_v, ssem).wait()

    def pass_body(h, _):
        hoff = h * _N_NODES

        pltpu.sync_copy(zeros_hbm, acc_s.at[pl.ds(sid * _ZCH, _ZCH)])
        plsc.subcore_barrier()

        _slab_load(0, 0)
        _slab_wait(0, hoff)
        pltpu.async_copy(emb_hbm.at[sidx_v.at[0, 0]], r0_v, gsem)

        def slab_iter(si, _):
            sb = si & 1

            @pl.when(si < _NSLABS - 1)
            def _():
                _slab_load(si + 1, sb ^ 1)

            for u in range(_SUP):
                _gwait()

                if u == _SUP - 1:
                    @pl.when(si < _NSLABS - 1)
                    def _():
                        _slab_wait(sb ^ 1, hoff)

                if u == _SUP - 1:
                    _swait()
                else:
                    @pl.when(si >= 1)
                    def _():
                        _swait()

                if u < _SUP - 1:
                    pltpu.async_copy(emb_hbm.at[sidx_v.at[sb, u + 1]],
                                     bufs[u + 1], gsem)
                else:
                    @pl.when(si < _NSLABS - 1)
                    def _():
                        pltpu.async_copy(emb_hbm.at[sidx_v.at[sb ^ 1, 0]],
                                         bufs[0], gsem)

                for t in range(6):
                    s = pl.ds(16 * t, 16)
                    lv = dslab_v[sb, u, s] - half_base
                    ok = (lv >= 0) & (lv < _HALF)
                    ldst_v[u, s] = jnp.where(ok, lv, _HALF)

                rbuf = bufs[u]

                def grp16(t, _):
                    vv = val_v[sb, u, pl.ds(16 * t, 16)]
                    for w in range(16):
                        e = t * 16 + w
                        v = vv[w]
                        for q in range(_DH // 16):
                            s = pl.ds(16 * q, 16)
                            rbuf[e, s] = rbuf[e, s] * v
                    return 0

                lax.fori_loop(0, _ECOL // 16, grp16, 0)
                pltpu.async_copy(rbuf, acc_s.at[ldst_v.at[u]], ssem, add=True)
            return 0

        lax.fori_loop(0, _NSLABS, slab_iter, 0)

        def drain(i, _):
            _swait()
            return 0

        lax.fori_loop(0, 4, drain, 0)
        plsc.subcore_barrier()

        r = sid * (_HALF // _NS)
        pltpu.sync_copy(
            acc_s.at[pl.ds(r, _HALF // _NS)],
            out_hbm.at[pl.ds(hoff + half_base + r, _HALF // _NS)])
        plsc.subcore_barrier()
        return 0

    lax.fori_loop(0, 2, pass_body, 0)


def _combine_body(a_ref, b_ref, c_ref, d_ref, o_ref):
    o_ref[...] = (a_ref[...] + b_ref[...] + c_ref[...] + d_ref[...]) * 0.25


_combine = pl.pallas_call(
    _combine_body,
    grid=(60,),
    in_specs=[pl.BlockSpec((2000, _DH), lambda i: (i, 0))] * 4,
    out_specs=pl.BlockSpec((2000, _DH), lambda i: (i, 0)),
    out_shape=jax.ShapeDtypeStruct((2 * _N_NODES, _DH), jnp.float32),
)


def kernel(W_user, W_item, spk_emb, edge_values, user_fea_index,
           item_fea_index, item_fea_mask, edge_index):
    f32 = jnp.float32
    i32 = jnp.int32
    ufi = user_fea_index.astype(i32).T.reshape(_NUCH * 5, _UCH)
    ifi = item_fea_index.astype(i32).T.reshape(_NICH * 6, _ICH)
    imask = item_fea_mask.astype(i32).T.reshape(_NICH * 6, _ICH)
    src = edge_index[0].astype(i32).reshape(_EROWS, _ECOL)
    dst = edge_index[1].astype(i32).reshape(_EROWS, _ECOL)
    val = edge_values.astype(f32).reshape(_EROWS, _ECOL)
    zeros = jnp.zeros((_ZCH, _DH), f32)

    e0 = _emb0_kernel(W_user.astype(f32), ufi, spk_emb.astype(f32),
                      W_item.astype(f32), ifi, imask)
    e1 = _layer_kernel(e0, src, dst, val, zeros)
    e2 = _layer_kernel(e1, src, dst, val, zeros)
    e3 = _layer_kernel(e2, src, dst, val, zeros)
    out = _combine(e0, e1, e2, e3)
    users = jnp.concatenate(
        [out[:_N_USERS], out[_N_NODES:_N_NODES + _N_USERS]], axis=1)
    items = jnp.concatenate(
        [out[_N_USERS:_N_NODES], out[_N_NODES + _N_USERS:]], axis=1)
    return users, items
